# baseline (speedup 1.0000x reference)
import functools

import jax
import jax.numpy as jnp
from jax import lax
from jax.experimental import pallas as pl
from jax.experimental.pallas import tpu as pltpu
from jax.experimental.pallas import tpu_sc as plsc

H = 16
NC = 2
NS = 16
NW = NC * NS
LANES = 128
ROWS_PER_CHUNK = 5
CHUNK = LANES * ROWS_PER_CHUNK
PIECE = 3200


def _blk(W):
    return jnp.kron(jnp.eye(8, dtype=jnp.float32), W.astype(jnp.float32))


def _blkb(b):
    return jnp.tile(b.astype(jnp.float32), 8).reshape(1, 8 * b.shape[0])


def _node_pre_body(cx, vx, wce, bce, wve, bve, wcs, wv2c, wvs,
                   s_c, hv2c, s_v):
    f32 = jnp.float32
    h_c0 = jnp.maximum(
        jnp.dot(cx[...], wce[...], preferred_element_type=f32) + bce[...], 0.0)
    h_v = jnp.maximum(
        jnp.dot(vx[...], wve[...], preferred_element_type=f32) + bve[...], 0.0)
    s_c[...] = jnp.dot(h_c0, wcs[...], preferred_element_type=f32)
    hv2c[...] = jnp.dot(h_v, wv2c[...], preferred_element_type=f32)
    s_v[...] = jnp.dot(h_v, wvs[...], preferred_element_type=f32)


def _node_pre(cons_x, var_x, W_ce, b_ce, W_ve, b_ve, W_cself, W_v2c, W_vself):
    n_cons = cons_x.shape[0]
    n_vars = var_x.shape[0]
    dc, dv = cons_x.shape[1], var_x.shape[1]
    nc8, nv8 = n_cons // 8, n_vars // 8
    out_shape = [
        jax.ShapeDtypeStruct((nc8, 8 * H), jnp.float32),
        jax.ShapeDtypeStruct((nv8, 8 * H), jnp.float32),
        jax.ShapeDtypeStruct((nv8, 8 * H), jnp.float32),
    ]
    return pl.pallas_call(_node_pre_body, out_shape=out_shape)(
        cons_x.reshape(nc8, 8 * dc), var_x.reshape(nv8, 8 * dv),
        _blk(W_ce), _blkb(b_ce), _blk(W_ve), _blkb(b_ve),
        _blk(W_cself), _blk(W_v2c), _blk(W_vself))


def _edge_pre_body(ea_t, wee, bee, wout, out8):
    f32 = jnp.float32
    dn = (((0,), (0,)), ((), ()))
    h_p = jnp.maximum(
        lax.dot_general(ea_t[...], wee[...], dn, preferred_element_type=f32)
        + bee[...], 0.0)
    o_p = jnp.dot(h_p, wout[...], preferred_element_type=f32)
    g = pl.program_id(0)
    for s in range(8):
        @pl.when(g % 8 == s)
        def _():
            out8[:, s * H:(s + 1) * H] = o_p


def _edge_pre_one(edge_attr, W_ee, b_ee, W_out):
    E, DE = edge_attr.shape
    assert E % PIECE == 0 and (E // 8) % PIECE == 0
    grid = (E // PIECE,)
    full = lambda shape: pl.BlockSpec(shape, lambda g: (0, 0))
    return pl.pallas_call(
        _edge_pre_body,
        grid=grid,
        in_specs=[
            pl.BlockSpec((DE, PIECE), lambda g: (0, g)),
            full((DE, H)), full((1, H)), full((H, H)),
        ],
        out_specs=pl.BlockSpec((PIECE, 8 * H), lambda g: (g // 8, 0)),
        out_shape=jax.ShapeDtypeStruct((E // 8, 8 * H), jnp.float32),
    )(edge_attr.T, W_ee, b_ee.reshape(1, H), W_out)


def _sc_message_pass(table, eidx3, gdim, elin, n_rows):
    E = eidx3.shape[1] * LANES
    n_chunks = E // CHUNK
    per_w = -(-n_chunks // NW)
    n_pad = -(-n_rows // (NS * 8)) * (NS * 8)
    rows_per_tile = n_pad // NS
    sdim = 1 - gdim

    @functools.partial(
        pl.kernel,
        out_type=jax.ShapeDtypeStruct((NC, n_pad, H), jnp.float32),
        mesh=plsc.VectorSubcoreMesh(core_axis_name="c", subcore_axis_name="s",
                                    num_cores=NC, num_subcores=NS),
        compiler_params=pltpu.CompilerParams(use_tc_tiling_on_sc=False,
                                             needs_layout_passes=False),
        scratch_types=[
            pltpu.VMEM((ROWS_PER_CHUNK, LANES), jnp.int32),
            pltpu.VMEM((ROWS_PER_CHUNK, LANES), jnp.int32),
            pltpu.VMEM((CHUNK, H), jnp.float32),
            pltpu.VMEM((CHUNK, H), jnp.float32),
            pltpu.VMEM_SHARED((n_pad, H), jnp.float32),
            pltpu.SemaphoreType.DMA,
        ],
    )
    def k(table_hbm, eidx_hbm, elin_hbm, out_hbm,
          gi_v, si_v, rows_v, el_v, acc_sh, sem):
        cid = lax.axis_index("c")
        sid = lax.axis_index("s")
        wid = sid * NC + cid

        def zrow(i, c):
            rows_v[i] = jnp.zeros((H,), jnp.float32)
            return c
        lax.fori_loop(0, CHUNK, zrow, 0)
        zbase = sid * rows_per_tile
        nfull = rows_per_tile // CHUNK
        for z in range(nfull):
            pltpu.sync_copy(rows_v, acc_sh.at[pl.ds(zbase + z * CHUNK, CHUNK)])
        rem = rows_per_tile - nfull * CHUNK
        if rem:
            pltpu.sync_copy(rows_v.at[pl.ds(0, rem)],
                            acc_sh.at[pl.ds(zbase + nfull * CHUNK, rem)])
        plsc.subcore_barrier()

        def chunk_body(t, c):
            ck = t * NW + wid

            @pl.when(ck < n_chunks)
            def _():
                roff = ck * ROWS_PER_CHUNK
                pltpu.sync_copy(eidx_hbm.at[gdim, pl.ds(roff, ROWS_PER_CHUNK)],
                                gi_v)
                pltpu.sync_copy(eidx_hbm.at[sdim, pl.ds(roff, ROWS_PER_CHUNK)],
                                si_v)
                cpp = PIECE // CHUNK
                g = ck // cpp
                row0 = (g // 8) * PIECE + (ck % cpp) * CHUNK
                lane0 = (g % 8) * H
                pltpu.sync_copy(
                    elin_hbm.at[pl.ds(row0, CHUNK), pl.ds(lane0, H)], el_v)
                descs = [
                    pltpu.async_copy(table_hbm.at[gi_v.at[j]],
                                     rows_v.at[pl.ds(j * LANES, LANES)], sem)
                    for j in range(ROWS_PER_CHUNK)
                ]
                for d in descs:
                    d.wait()

                def fuse(r, cc):
                    for kk in range(8):
                        e = r * 8 + kk
                        rows_v[e] = jnp.maximum(rows_v[e] + el_v[e], 0.0)
                    return cc
                lax.fori_loop(0, CHUNK // 8, fuse, 0)
                for j in range(ROWS_PER_CHUNK):
                    pltpu.sync_copy(rows_v.at[pl.ds(j * LANES, LANES)],
                                    acc_sh.at[si_v.at[j]], add=True)
            return c
        lax.fori_loop(0, per_w, chunk_body, 0)

        plsc.subcore_barrier()
        pltpu.sync_copy(acc_sh.at[pl.ds(zbase, rows_per_tile)],
                        out_hbm.at[cid].at[pl.ds(zbase, rows_per_tile)])

    return k(table, eidx3, elin)


def _combine_c_body(n8, sc, agg, bc, wc2v, hc2v, sum_c):
    h_c = jnp.maximum(sc[...] + agg[0, :n8] + agg[1, :n8] + bc[...], 0.0)
    hc2v[...] = jnp.dot(h_c, wc2v[...], preferred_element_type=jnp.float32)
    sum_c[...] = jnp.sum(h_c, axis=0, keepdims=True)


def _combine_c(s_c8, agg_c8, b_c, W_c2v):
    n8 = s_c8.shape[0]
    return pl.pallas_call(
        functools.partial(_combine_c_body, n8),
        out_shape=[jax.ShapeDtypeStruct((n8, 8 * H), jnp.float32),
                   jax.ShapeDtypeStruct((1, 8 * H), jnp.float32)],
    )(s_c8, agg_c8, _blkb(b_c), _blk(W_c2v))


def _combine_v_body(n8, sv, agg, bv, sum_v):
    h_v = jnp.maximum(sv[...] + agg[0, :n8] + agg[1, :n8] + bv[...], 0.0)
    sum_v[...] = jnp.sum(h_v, axis=0, keepdims=True)


def _combine_v(s_v8, agg_v8, b_v):
    n8 = s_v8.shape[0]
    return pl.pallas_call(
        functools.partial(_combine_v_body, n8),
        out_shape=jax.ShapeDtypeStruct((1, 8 * H), jnp.float32),
    )(s_v8, agg_v8, _blkb(b_v))


def kernel(cons_x, edge_index, edge_attr, var_x, W_ce, b_ce, W_ve, b_ve,
           W_ee, b_ee, W_v2c, W_e2c, W_cself, b_c, W_c2v, W_e2v, W_vself,
           b_v, W_head, b_head):
    n_cons = cons_x.shape[0]
    n_vars = var_x.shape[0]
    E = edge_attr.shape[0]
    assert E % CHUNK == 0 and n_cons % NS == 0 and n_vars % NS == 0

    eidx3 = edge_index.astype(jnp.int32).reshape(2, E // LANES, LANES)

    s_c8, hv2c8, s_v8 = _node_pre(cons_x, var_x, W_ce, b_ce, W_ve, b_ve,
                                  W_cself, W_v2c, W_vself)
    ec = _edge_pre_one(edge_attr, W_ee, b_ee, W_e2c)
    ev = _edge_pre_one(edge_attr, W_ee, b_ee, W_e2v)

    agg_c = _sc_message_pass(hv2c8.reshape(n_vars, H), eidx3, 1, ec, n_cons)
    agg_c8 = agg_c.reshape(NC, agg_c.shape[1] // 8, 8 * H)
    hc2v8, sum_c = _combine_c(s_c8, agg_c8, b_c, W_c2v)

    agg_v = _sc_message_pass(hc2v8.reshape(n_cons, H), eidx3, 0, ev, n_vars)
    agg_v8 = agg_v.reshape(NC, agg_v.shape[1] // 8, 8 * H)
    sum_v = _combine_v(s_v8, agg_v8, b_v)

    z = (sum_c.reshape(8, H).sum(0) + sum_v.reshape(8, H).sum(0)) \
        / jnp.float32(n_cons + n_vars)
    return z @ W_head + b_head


# speedup vs baseline: 1.2703x; 1.2703x over previous
import functools

import jax
import jax.numpy as jnp
from jax import lax
from jax.experimental import pallas as pl
from jax.experimental.pallas import tpu as pltpu
from jax.experimental.pallas import tpu_sc as plsc

H = 16
NC = 2
NS = 16
NW = NC * NS
LANES = 128
ROWS_PER_CHUNK = 5
CHUNK = LANES * ROWS_PER_CHUNK
PIECE = 16000


def _blk(W):
    return jnp.kron(jnp.eye(8, dtype=jnp.float32), W.astype(jnp.float32))


def _blkb(b):
    return jnp.tile(b.astype(jnp.float32), 8).reshape(1, 8 * b.shape[0])


def _node_pre_body(cx, vx, wce, bce, wve, bve, wcs, wv2c, wvs,
                   s_c, hv2c, s_v):
    f32 = jnp.float32
    h_c0 = jnp.maximum(
        jnp.dot(cx[...], wce[...], preferred_element_type=f32) + bce[...], 0.0)
    h_v = jnp.maximum(
        jnp.dot(vx[...], wve[...], preferred_element_type=f32) + bve[...], 0.0)
    s_c[...] = jnp.dot(h_c0, wcs[...], preferred_element_type=f32)
    hv2c[...] = jnp.dot(h_v, wv2c[...], preferred_element_type=f32)
    s_v[...] = jnp.dot(h_v, wvs[...], preferred_element_type=f32)


def _node_pre(cons_x, var_x, W_ce, b_ce, W_ve, b_ve, W_cself, W_v2c, W_vself):
    n_cons = cons_x.shape[0]
    n_vars = var_x.shape[0]
    dc, dv = cons_x.shape[1], var_x.shape[1]
    nc8, nv8 = n_cons // 8, n_vars // 8
    out_shape = [
        jax.ShapeDtypeStruct((nc8, 8 * H), jnp.float32),
        jax.ShapeDtypeStruct((nv8, 8 * H), jnp.float32),
        jax.ShapeDtypeStruct((nv8, 8 * H), jnp.float32),
    ]
    return pl.pallas_call(_node_pre_body, out_shape=out_shape)(
        cons_x.reshape(nc8, 8 * dc), var_x.reshape(nv8, 8 * dv),
        _blk(W_ce), _blkb(b_ce), _blk(W_ve), _blkb(b_ve),
        _blk(W_cself), _blk(W_v2c), _blk(W_vself))


def _edge_pre_body(ea_t, wee, bee, wout, out8):
    f32 = jnp.float32
    dn = (((0,), (0,)), ((), ()))
    h_p = jnp.maximum(
        lax.dot_general(ea_t[...], wee[...], dn, preferred_element_type=f32)
        + bee[...], 0.0)
    o_p = jnp.dot(h_p, wout[...], preferred_element_type=f32)
    g = pl.program_id(0)
    for s in range(8):
        @pl.when(g % 8 == s)
        def _():
            out8[:, s * H:(s + 1) * H] = o_p


def _edge_pre_one(edge_attr, W_ee, b_ee, W_out):
    E, DE = edge_attr.shape
    assert E % PIECE == 0 and (E // 8) % PIECE == 0
    grid = (E // PIECE,)
    full = lambda shape: pl.BlockSpec(shape, lambda g: (0, 0))
    return pl.pallas_call(
        _edge_pre_body,
        grid=grid,
        in_specs=[
            pl.BlockSpec((DE, PIECE), lambda g: (0, g)),
            full((DE, H)), full((1, H)), full((H, H)),
        ],
        out_specs=pl.BlockSpec((PIECE, 8 * H), lambda g: (g // 8, 0)),
        out_shape=jax.ShapeDtypeStruct((E // 8, 8 * H), jnp.float32),
    )(edge_attr.T, W_ee, b_ee.reshape(1, H), W_out)


def _sc_message_pass(table, eidx3, gdim, elin, n_rows):
    E = eidx3.shape[1] * LANES
    n_chunks = E // CHUNK
    per_w = -(-n_chunks // NW)
    n_pad = -(-n_rows // (NS * 8)) * (NS * 8)
    rows_per_tile = n_pad // NS
    sdim = 1 - gdim

    @functools.partial(
        pl.kernel,
        out_type=jax.ShapeDtypeStruct((NC, n_pad, H), jnp.float32),
        mesh=plsc.VectorSubcoreMesh(core_axis_name="c", subcore_axis_name="s",
                                    num_cores=NC, num_subcores=NS),
        compiler_params=pltpu.CompilerParams(use_tc_tiling_on_sc=False,
                                             needs_layout_passes=False),
        scratch_types=[
            pltpu.VMEM((ROWS_PER_CHUNK, LANES), jnp.int32),
            pltpu.VMEM((ROWS_PER_CHUNK, LANES), jnp.int32),
            pltpu.VMEM((CHUNK, H), jnp.float32),
            pltpu.VMEM((CHUNK, H), jnp.float32),
            pltpu.VMEM_SHARED((n_pad, H), jnp.float32),
            pltpu.SemaphoreType.DMA,
        ],
    )
    def k(table_hbm, eidx_hbm, elin_hbm, out_hbm,
          gi_v, si_v, rows_v, el_v, acc_sh, sem):
        cid = lax.axis_index("c")
        sid = lax.axis_index("s")
        wid = sid * NC + cid

        def zrow(i, c):
            rows_v[i] = jnp.zeros((H,), jnp.float32)
            return c
        lax.fori_loop(0, CHUNK, zrow, 0)
        zbase = sid * rows_per_tile
        nfull = rows_per_tile // CHUNK
        for z in range(nfull):
            pltpu.sync_copy(rows_v, acc_sh.at[pl.ds(zbase + z * CHUNK, CHUNK)])
        rem = rows_per_tile - nfull * CHUNK
        if rem:
            pltpu.sync_copy(rows_v.at[pl.ds(0, rem)],
                            acc_sh.at[pl.ds(zbase + nfull * CHUNK, rem)])
        plsc.subcore_barrier()

        def chunk_body(t, c):
            ck = t * NW + wid

            @pl.when(ck < n_chunks)
            def _():
                roff = ck * ROWS_PER_CHUNK
                pltpu.sync_copy(eidx_hbm.at[gdim, pl.ds(roff, ROWS_PER_CHUNK)],
                                gi_v)
                pltpu.sync_copy(eidx_hbm.at[sdim, pl.ds(roff, ROWS_PER_CHUNK)],
                                si_v)
                cpp = PIECE // CHUNK
                g = ck // cpp
                row0 = (g // 8) * PIECE + (ck % cpp) * CHUNK
                lane0 = (g % 8) * H
                pltpu.sync_copy(
                    elin_hbm.at[pl.ds(row0, CHUNK), pl.ds(lane0, H)], el_v)
                descs = [
                    pltpu.async_copy(table_hbm.at[gi_v.at[j]],
                                     rows_v.at[pl.ds(j * LANES, LANES)], sem)
                    for j in range(ROWS_PER_CHUNK)
                ]
                for d in descs:
                    d.wait()

                def fuse(r, cc):
                    for kk in range(8):
                        e = r * 8 + kk
                        rows_v[e] = jnp.maximum(rows_v[e] + el_v[e], 0.0)
                    return cc
                lax.fori_loop(0, CHUNK // 8, fuse, 0)
                for j in range(ROWS_PER_CHUNK):
                    pltpu.sync_copy(rows_v.at[pl.ds(j * LANES, LANES)],
                                    acc_sh.at[si_v.at[j]], add=True)
            return c
        lax.fori_loop(0, per_w, chunk_body, 0)

        plsc.subcore_barrier()
        pltpu.sync_copy(acc_sh.at[pl.ds(zbase, rows_per_tile)],
                        out_hbm.at[cid].at[pl.ds(zbase, rows_per_tile)])

    return k(table, eidx3, elin)


def _combine_c_body(n8, sc, agg, bc, wc2v, hc2v, sum_c):
    h_c = jnp.maximum(sc[...] + agg[0, :n8] + agg[1, :n8] + bc[...], 0.0)
    hc2v[...] = jnp.dot(h_c, wc2v[...], preferred_element_type=jnp.float32)
    sum_c[...] = jnp.sum(h_c, axis=0, keepdims=True)


def _combine_c(s_c8, agg_c8, b_c, W_c2v):
    n8 = s_c8.shape[0]
    return pl.pallas_call(
        functools.partial(_combine_c_body, n8),
        out_shape=[jax.ShapeDtypeStruct((n8, 8 * H), jnp.float32),
                   jax.ShapeDtypeStruct((1, 8 * H), jnp.float32)],
    )(s_c8, agg_c8, _blkb(b_c), _blk(W_c2v))


def _combine_v_body(n8, sv, agg, bv, sum_v):
    h_v = jnp.maximum(sv[...] + agg[0, :n8] + agg[1, :n8] + bv[...], 0.0)
    sum_v[...] = jnp.sum(h_v, axis=0, keepdims=True)


def _combine_v(s_v8, agg_v8, b_v):
    n8 = s_v8.shape[0]
    return pl.pallas_call(
        functools.partial(_combine_v_body, n8),
        out_shape=jax.ShapeDtypeStruct((1, 8 * H), jnp.float32),
    )(s_v8, agg_v8, _blkb(b_v))


def kernel(cons_x, edge_index, edge_attr, var_x, W_ce, b_ce, W_ve, b_ve,
           W_ee, b_ee, W_v2c, W_e2c, W_cself, b_c, W_c2v, W_e2v, W_vself,
           b_v, W_head, b_head):
    n_cons = cons_x.shape[0]
    n_vars = var_x.shape[0]
    E = edge_attr.shape[0]
    assert E % CHUNK == 0 and n_cons % NS == 0 and n_vars % NS == 0

    eidx3 = edge_index.astype(jnp.int32).reshape(2, E // LANES, LANES)

    s_c8, hv2c8, s_v8 = _node_pre(cons_x, var_x, W_ce, b_ce, W_ve, b_ve,
                                  W_cself, W_v2c, W_vself)
    ec = _edge_pre_one(edge_attr, W_ee, b_ee, W_e2c)
    ev = _edge_pre_one(edge_attr, W_ee, b_ee, W_e2v)

    agg_c = _sc_message_pass(hv2c8.reshape(n_vars, H), eidx3, 1, ec, n_cons)
    agg_c8 = agg_c.reshape(NC, agg_c.shape[1] // 8, 8 * H)
    hc2v8, sum_c = _combine_c(s_c8, agg_c8, b_c, W_c2v)

    agg_v = _sc_message_pass(hc2v8.reshape(n_cons, H), eidx3, 0, ev, n_vars)
    agg_v8 = agg_v.reshape(NC, agg_v.shape[1] // 8, 8 * H)
    sum_v = _combine_v(s_v8, agg_v8, b_v)

    z = (sum_c.reshape(8, H).sum(0) + sum_v.reshape(8, H).sum(0)) \
        / jnp.float32(n_cons + n_vars)
    return z @ W_head + b_head
